# baseline (speedup 1.0000x reference)
import functools

import jax
import jax.numpy as jnp
from jax import lax
from jax.experimental import pallas as pl
from jax.experimental.pallas import tpu as pltpu
from jax.experimental.pallas import tpu_sc as plsc

N = 10000
D = 128
NC = 2
NS = 16
NW = NC * NS
N_PAD = 10240
STRIPE = N_PAD // NS
E = 320000 + N
K = 128
G = 82
G_AL = G + 1
E_PAD = NW * K * G
DEGC = 16
RB = 2000
NB = N // RB

_mesh = plsc.VectorSubcoreMesh(core_axis_name="c", subcore_axis_name="s")


@functools.partial(
    pl.kernel,
    out_type=jax.ShapeDtypeStruct((NC, N_PAD, DEGC), jnp.float32),
    mesh=_mesh,
    scratch_types=[
        pltpu.VMEM((G_AL, K), jnp.int32),
        pltpu.VMEM((K, DEGC), jnp.float32),
        pltpu.VMEM_SHARED((N_PAD, DEGC), jnp.float32),
    ],
)
def _sc_deg(dst_hbm, ones_hbm, zdeg_hbm, out_hbm, dst_v, ones_v, acc_sh):
    c = lax.axis_index("c")
    s = lax.axis_index("s")
    wid = c * NS + s
    pltpu.sync_copy(dst_hbm.at[wid], dst_v)
    pltpu.sync_copy(ones_hbm, ones_v)
    pltpu.sync_copy(zdeg_hbm, acc_sh.at[pl.ds(s * STRIPE, STRIPE)])
    plsc.subcore_barrier()

    def body(g, carry):
        pltpu.sync_copy(ones_v, acc_sh.at[dst_v.at[g]], add=True)
        return carry

    lax.fori_loop(0, G, body, 0)
    plsc.subcore_barrier()
    pltpu.sync_copy(acc_sh.at[pl.ds(s * STRIPE, STRIPE)],
                    out_hbm.at[c, pl.ds(s * STRIPE, STRIPE)])


@functools.partial(
    pl.kernel,
    out_type=jax.ShapeDtypeStruct((NC, N_PAD, D), jnp.float32),
    mesh=_mesh,
    scratch_types=[
        pltpu.VMEM((K,), jnp.int32),
        pltpu.VMEM((K,), jnp.int32),
        pltpu.VMEM((K, D), jnp.float32),
        pltpu.VMEM_SHARED((N_PAD, D), jnp.float32),
        pltpu.SemaphoreType.DMA,
    ],
)
def _sc_agg(h_hbm, src_hbm, dst_hbm, zrow_hbm, out_hbm,
            src_v, dst_v, rows_v, acc_sh, sem):
    c = lax.axis_index("c")
    s = lax.axis_index("s")
    wid = c * NS + s
    pltpu.sync_copy(zrow_hbm, acc_sh.at[pl.ds(s * STRIPE, STRIPE)])
    plsc.subcore_barrier()
    base0 = wid * (G * K)

    def body(g, carry):
        base = base0 + g * K
        pltpu.sync_copy(src_hbm.at[pl.ds(base, K)], src_v)
        pltpu.sync_copy(dst_hbm.at[pl.ds(base, K)], dst_v)
        pltpu.async_copy(h_hbm.at[src_v], rows_v, sem).wait()
        pltpu.sync_copy(rows_v, acc_sh.at[dst_v], add=True)
        return carry

    lax.fori_loop(0, G, body, 0)
    plsc.subcore_barrier()
    pltpu.sync_copy(acc_sh.at[pl.ds(s * STRIPE, STRIPE)],
                    out_hbm.at[c, pl.ds(s * STRIPE, STRIPE)])


def _dinv(degp_ref):
    return lax.rsqrt(degp_ref[0, :, :1] + degp_ref[1, :, :1])


def _tc_in_body(x_ref, w_ref, degp_ref, o_ref):
    o_ref[...] = _dinv(degp_ref) * jnp.dot(
        x_ref[...], w_ref[...], preferred_element_type=jnp.float32)


_tc_in = pl.pallas_call(
    _tc_in_body,
    grid=(NB,),
    in_specs=[
        pl.BlockSpec((RB, D), lambda i: (i, 0)),
        pl.BlockSpec((D, D), lambda i: (0, 0)),
        pl.BlockSpec((NC, RB, DEGC), lambda i: (0, i, 0)),
    ],
    out_specs=pl.BlockSpec((RB, D), lambda i: (i, 0)),
    out_shape=jax.ShapeDtypeStruct((N, D), jnp.float32),
)


def _tc_mid_body(p_ref, degp_ref, b1_ref, w2_ref, o_ref):
    dinv = _dinv(degp_ref)
    h2 = jnp.maximum(dinv * (p_ref[0] + p_ref[1]) + b1_ref[...], 0.0)
    o_ref[...] = dinv * jnp.dot(h2, w2_ref[...],
                                preferred_element_type=jnp.float32)


_tc_mid = pl.pallas_call(
    _tc_mid_body,
    grid=(NB,),
    in_specs=[
        pl.BlockSpec((NC, RB, D), lambda i: (0, i, 0)),
        pl.BlockSpec((NC, RB, DEGC), lambda i: (0, i, 0)),
        pl.BlockSpec((1, D), lambda i: (0, 0)),
        pl.BlockSpec((D, D), lambda i: (0, 0)),
    ],
    out_specs=pl.BlockSpec((RB, D), lambda i: (i, 0)),
    out_shape=jax.ShapeDtypeStruct((N, D), jnp.float32),
)


def _tc_out_body(q_ref, degp_ref, b2_ref, o_ref):
    o_ref[...] = _dinv(degp_ref) * (q_ref[0] + q_ref[1]) + b2_ref[...]


_tc_out = pl.pallas_call(
    _tc_out_body,
    grid=(NB,),
    in_specs=[
        pl.BlockSpec((NC, RB, D), lambda i: (0, i, 0)),
        pl.BlockSpec((NC, RB, DEGC), lambda i: (0, i, 0)),
        pl.BlockSpec((1, D), lambda i: (0, 0)),
    ],
    out_specs=pl.BlockSpec((RB, D), lambda i: (i, 0)),
    out_shape=jax.ShapeDtypeStruct((N, D), jnp.float32),
)


def kernel(x, edge_index, W1, b1, W2, b2):
    ei = edge_index.astype(jnp.int32)
    loop = jnp.arange(N, dtype=jnp.int32)
    pad = E_PAD - E
    src = jnp.concatenate([ei[0], loop, jnp.zeros((pad,), jnp.int32)])
    dst = jnp.concatenate([ei[1], loop, jnp.full((pad,), N, jnp.int32)])
    dst3 = dst.reshape(NW, G, K)
    dst3 = jnp.concatenate(
        [dst3, jnp.full((NW, G_AL - G, K), N, jnp.int32)], axis=1)
    ones_blk = jnp.ones((K, DEGC), jnp.float32)
    zdeg = jnp.zeros((STRIPE, DEGC), jnp.float32)
    zrow = jnp.zeros((STRIPE, D), jnp.float32)

    degp = _sc_deg(dst3, ones_blk, zdeg)
    h1 = _tc_in(x, W1, degp)
    p = _sc_agg(h1, src, dst, zrow)
    h3 = _tc_mid(p, degp, b1.reshape(1, D), W2)
    q = _sc_agg(h3, src, dst, zrow)
    return _tc_out(q, degp, b2.reshape(1, D))


# speedup vs baseline: 1.4570x; 1.4570x over previous
import functools

import jax
import jax.numpy as jnp
from jax import lax
from jax.experimental import pallas as pl
from jax.experimental.pallas import tpu as pltpu
from jax.experimental.pallas import tpu_sc as plsc

N = 10000
D = 128
NC = 2
NS = 16
NW = NC * NS
N_PAD = 10240
STRIPE = N_PAD // NS
E = 320000 + N
K = 128
G = 81
G_AL = G + 1
E_PAD = NW * K * G
DEGC = 16
RB = 2000
NB = N // RB

_mesh = plsc.VectorSubcoreMesh(core_axis_name="c", subcore_axis_name="s")


@functools.partial(
    pl.kernel,
    out_type=jax.ShapeDtypeStruct((NC, N_PAD, DEGC), jnp.float32),
    mesh=_mesh,
    scratch_types=[
        pltpu.VMEM((K,), jnp.int32),
        pltpu.VMEM((K, DEGC), jnp.float32),
        pltpu.VMEM_SHARED((N_PAD, DEGC), jnp.float32),
    ],
)
def _sc_deg(dst_hbm, ones_hbm, zdeg_hbm, out_hbm, dst_v, ones_v, acc_sh):
    c = lax.axis_index("c")
    s = lax.axis_index("s")
    wid = c * NS + s
    pltpu.sync_copy(ones_hbm, ones_v)
    pltpu.sync_copy(zdeg_hbm, acc_sh.at[pl.ds(s * STRIPE, STRIPE)])
    plsc.subcore_barrier()
    base0 = wid * (G * K)

    def body(g, carry):
        pltpu.sync_copy(dst_hbm.at[pl.ds(base0 + g * K, K)], dst_v)
        pltpu.sync_copy(ones_v, acc_sh.at[dst_v], add=True)
        return carry

    lax.fori_loop(0, G, body, 0)
    plsc.subcore_barrier()
    pltpu.sync_copy(acc_sh.at[pl.ds(s * STRIPE, STRIPE)],
                    out_hbm.at[c, pl.ds(s * STRIPE, STRIPE)])


@functools.partial(
    pl.kernel,
    out_type=jax.ShapeDtypeStruct((NC, N_PAD, D), jnp.float32),
    mesh=_mesh,
    scratch_types=[
        pltpu.VMEM((K,), jnp.int32),
        pltpu.VMEM((K,), jnp.int32),
        pltpu.VMEM((K, D), jnp.float32),
        pltpu.VMEM_SHARED((N_PAD, D), jnp.float32),
        pltpu.SemaphoreType.DMA,
    ],
)
def _sc_agg(h_hbm, src_hbm, dst_hbm, zrow_hbm, out_hbm,
            src_v, dst_v, rows_v, acc_sh, sem):
    c = lax.axis_index("c")
    s = lax.axis_index("s")
    wid = c * NS + s
    pltpu.sync_copy(zrow_hbm, acc_sh.at[pl.ds(s * STRIPE, STRIPE)])
    plsc.subcore_barrier()
    base0 = wid * (G * K)

    def body(g, carry):
        base = base0 + g * K
        pltpu.sync_copy(src_hbm.at[pl.ds(base, K)], src_v)
        pltpu.sync_copy(dst_hbm.at[pl.ds(base, K)], dst_v)
        pltpu.async_copy(h_hbm.at[src_v], rows_v, sem).wait()
        pltpu.sync_copy(rows_v, acc_sh.at[dst_v], add=True)
        return carry

    lax.fori_loop(0, G, body, 0)
    plsc.subcore_barrier()
    pltpu.sync_copy(acc_sh.at[pl.ds(s * STRIPE, STRIPE)],
                    out_hbm.at[c, pl.ds(s * STRIPE, STRIPE)])


def _dinv(degp_ref):
    return lax.rsqrt(degp_ref[0, :, :1] + degp_ref[1, :, :1])


def _tc_in_body(x_ref, w_ref, degp_ref, o_ref):
    o_ref[...] = _dinv(degp_ref) * jnp.dot(
        x_ref[...], w_ref[...], preferred_element_type=jnp.float32)


_tc_in = pl.pallas_call(
    _tc_in_body,
    grid=(NB,),
    in_specs=[
        pl.BlockSpec((RB, D), lambda i: (i, 0)),
        pl.BlockSpec((D, D), lambda i: (0, 0)),
        pl.BlockSpec((NC, RB, DEGC), lambda i: (0, i, 0)),
    ],
    out_specs=pl.BlockSpec((RB, D), lambda i: (i, 0)),
    out_shape=jax.ShapeDtypeStruct((N, D), jnp.float32),
)


def _tc_mid_body(p_ref, degp_ref, b1_ref, w2_ref, o_ref):
    dinv = _dinv(degp_ref)
    h2 = jnp.maximum(dinv * (p_ref[0] + p_ref[1]) + b1_ref[...], 0.0)
    o_ref[...] = dinv * jnp.dot(h2, w2_ref[...],
                                preferred_element_type=jnp.float32)


_tc_mid = pl.pallas_call(
    _tc_mid_body,
    grid=(NB,),
    in_specs=[
        pl.BlockSpec((NC, RB, D), lambda i: (0, i, 0)),
        pl.BlockSpec((NC, RB, DEGC), lambda i: (0, i, 0)),
        pl.BlockSpec((1, D), lambda i: (0, 0)),
        pl.BlockSpec((D, D), lambda i: (0, 0)),
    ],
    out_specs=pl.BlockSpec((RB, D), lambda i: (i, 0)),
    out_shape=jax.ShapeDtypeStruct((N, D), jnp.float32),
)


def _tc_out_body(q_ref, degp_ref, b2_ref, o_ref):
    o_ref[...] = _dinv(degp_ref) * (q_ref[0] + q_ref[1]) + b2_ref[...]


_tc_out = pl.pallas_call(
    _tc_out_body,
    grid=(NB,),
    in_specs=[
        pl.BlockSpec((NC, RB, D), lambda i: (0, i, 0)),
        pl.BlockSpec((NC, RB, DEGC), lambda i: (0, i, 0)),
        pl.BlockSpec((1, D), lambda i: (0, 0)),
    ],
    out_specs=pl.BlockSpec((RB, D), lambda i: (i, 0)),
    out_shape=jax.ShapeDtypeStruct((N, D), jnp.float32),
)


def kernel(x, edge_index, W1, b1, W2, b2):
    ei = edge_index.astype(jnp.int32)
    loop = jnp.arange(N, dtype=jnp.int32)
    pad = E_PAD - E
    src = jnp.concatenate([ei[0], loop, jnp.zeros((pad,), jnp.int32)])
    dst = jnp.concatenate([ei[1], loop, jnp.full((pad,), N, jnp.int32)])
    ones_blk = jnp.ones((K, DEGC), jnp.float32)
    zdeg = jnp.zeros((STRIPE, DEGC), jnp.float32)
    zrow = jnp.zeros((STRIPE, D), jnp.float32)

    degp = _sc_deg(dst, ones_blk, zdeg)
    h1 = _tc_in(x, W1, degp)
    p = _sc_agg(h1, src, dst, zrow)
    h3 = _tc_mid(p, degp, b1.reshape(1, D), W2)
    q = _sc_agg(h3, src, dst, zrow)
    return _tc_out(q, degp, b2.reshape(1, D))


# speedup vs baseline: 1.6068x; 1.1028x over previous
import functools

import jax
import jax.numpy as jnp
from jax import lax
from jax.experimental import pallas as pl
from jax.experimental.pallas import tpu as pltpu
from jax.experimental.pallas import tpu_sc as plsc

N = 10000
D = 128
NC = 2
NS = 16
NW = NC * NS
N_PAD = 10240
STRIPE = N_PAD // NS
E = 320000 + N
K = 128
G = 81
G_AL = G + 1
E_PAD = NW * K * G
DEGC = 16
RB = 2000
NB = N // RB

_mesh = plsc.VectorSubcoreMesh(core_axis_name="c", subcore_axis_name="s")


@functools.partial(
    pl.kernel,
    out_type=jax.ShapeDtypeStruct((NC, N_PAD, DEGC), jnp.float32),
    mesh=_mesh,
    scratch_types=[
        pltpu.VMEM((K,), jnp.int32),
        pltpu.VMEM((K, DEGC), jnp.float32),
        pltpu.VMEM_SHARED((N_PAD, DEGC), jnp.float32),
    ],
)
def _sc_deg(dst_hbm, ones_hbm, zdeg_hbm, out_hbm, dst_v, ones_v, acc_sh):
    c = lax.axis_index("c")
    s = lax.axis_index("s")
    wid = c * NS + s
    pltpu.sync_copy(ones_hbm, ones_v)
    pltpu.sync_copy(zdeg_hbm, acc_sh.at[pl.ds(s * STRIPE, STRIPE)])
    plsc.subcore_barrier()
    base0 = wid * (G * K)

    def body(g, carry):
        pltpu.sync_copy(dst_hbm.at[pl.ds(base0 + g * K, K)], dst_v)
        pltpu.sync_copy(ones_v, acc_sh.at[dst_v], add=True)
        return carry

    lax.fori_loop(0, G, body, 0)
    plsc.subcore_barrier()
    pltpu.sync_copy(acc_sh.at[pl.ds(s * STRIPE, STRIPE)],
                    out_hbm.at[c, pl.ds(s * STRIPE, STRIPE)])


@functools.partial(
    pl.kernel,
    out_type=jax.ShapeDtypeStruct((NC, N_PAD, D), jnp.float32),
    mesh=_mesh,
    scratch_types=[
        pltpu.VMEM((K,), jnp.int32),
        pltpu.VMEM((K,), jnp.int32),
        pltpu.VMEM((K, D), jnp.float32),
        pltpu.VMEM_SHARED((N_PAD, D), jnp.float32),
        pltpu.SemaphoreType.DMA,
    ],
)
def _sc_agg(h_hbm, src_hbm, dst_hbm, zrow_hbm, out_hbm,
            src_v, dst_v, rows_v, acc_sh, sem):
    c = lax.axis_index("c")
    s = lax.axis_index("s")
    wid = c * NS + s
    pltpu.sync_copy(zrow_hbm, acc_sh.at[pl.ds(s * STRIPE, STRIPE)])
    plsc.subcore_barrier()
    base0 = wid * (G * K)

    def body(g, carry):
        base = base0 + g * K
        pltpu.sync_copy(src_hbm.at[pl.ds(base, K)], src_v)
        gather = pltpu.async_copy(h_hbm.at[src_v], rows_v, sem)
        pltpu.sync_copy(dst_hbm.at[pl.ds(base, K)], dst_v)
        gather.wait()
        pltpu.sync_copy(rows_v, acc_sh.at[dst_v], add=True)
        return carry

    lax.fori_loop(0, G, body, 0)
    plsc.subcore_barrier()
    pltpu.sync_copy(acc_sh.at[pl.ds(s * STRIPE, STRIPE)],
                    out_hbm.at[c, pl.ds(s * STRIPE, STRIPE)])


def _dinv(degp_ref):
    return lax.rsqrt(degp_ref[0, :, :1] + degp_ref[1, :, :1])


def _tc_in_body(x_ref, w_ref, degp_ref, o_ref):
    o_ref[...] = _dinv(degp_ref) * jnp.dot(
        x_ref[...], w_ref[...], preferred_element_type=jnp.float32)


_tc_in = pl.pallas_call(
    _tc_in_body,
    grid=(NB,),
    in_specs=[
        pl.BlockSpec((RB, D), lambda i: (i, 0)),
        pl.BlockSpec((D, D), lambda i: (0, 0)),
        pl.BlockSpec((NC, RB, DEGC), lambda i: (0, i, 0)),
    ],
    out_specs=pl.BlockSpec((RB, D), lambda i: (i, 0)),
    out_shape=jax.ShapeDtypeStruct((N, D), jnp.float32),
)


def _tc_mid_body(p_ref, degp_ref, b1_ref, w2_ref, o_ref):
    dinv = _dinv(degp_ref)
    h2 = jnp.maximum(dinv * (p_ref[0] + p_ref[1]) + b1_ref[...], 0.0)
    o_ref[...] = dinv * jnp.dot(h2, w2_ref[...],
                                preferred_element_type=jnp.float32)


_tc_mid = pl.pallas_call(
    _tc_mid_body,
    grid=(NB,),
    in_specs=[
        pl.BlockSpec((NC, RB, D), lambda i: (0, i, 0)),
        pl.BlockSpec((NC, RB, DEGC), lambda i: (0, i, 0)),
        pl.BlockSpec((1, D), lambda i: (0, 0)),
        pl.BlockSpec((D, D), lambda i: (0, 0)),
    ],
    out_specs=pl.BlockSpec((RB, D), lambda i: (i, 0)),
    out_shape=jax.ShapeDtypeStruct((N, D), jnp.float32),
)


def _tc_out_body(q_ref, degp_ref, b2_ref, o_ref):
    o_ref[...] = _dinv(degp_ref) * (q_ref[0] + q_ref[1]) + b2_ref[...]


_tc_out = pl.pallas_call(
    _tc_out_body,
    grid=(NB,),
    in_specs=[
        pl.BlockSpec((NC, RB, D), lambda i: (0, i, 0)),
        pl.BlockSpec((NC, RB, DEGC), lambda i: (0, i, 0)),
        pl.BlockSpec((1, D), lambda i: (0, 0)),
    ],
    out_specs=pl.BlockSpec((RB, D), lambda i: (i, 0)),
    out_shape=jax.ShapeDtypeStruct((N, D), jnp.float32),
)


def kernel(x, edge_index, W1, b1, W2, b2):
    ei = edge_index.astype(jnp.int32)
    loop = jnp.arange(N, dtype=jnp.int32)
    pad = E_PAD - E
    src = jnp.concatenate([ei[0], loop, jnp.zeros((pad,), jnp.int32)])
    dst = jnp.concatenate([ei[1], loop, jnp.full((pad,), N, jnp.int32)])
    ones_blk = jnp.ones((K, DEGC), jnp.float32)
    zdeg = jnp.zeros((STRIPE, DEGC), jnp.float32)
    zrow = jnp.zeros((STRIPE, D), jnp.float32)

    degp = _sc_deg(dst, ones_blk, zdeg)
    h1 = _tc_in(x, W1, degp)
    p = _sc_agg(h1, src, dst, zrow)
    h3 = _tc_mid(p, degp, b1.reshape(1, D), W2)
    q = _sc_agg(h3, src, dst, zrow)
    return _tc_out(q, degp, b2.reshape(1, D))


# speedup vs baseline: 1.8079x; 1.1251x over previous
import functools

import jax
import jax.numpy as jnp
from jax import lax
from jax.experimental import pallas as pl
from jax.experimental.pallas import tpu as pltpu
from jax.experimental.pallas import tpu_sc as plsc

N = 10000
D = 128
NC = 2
NS = 16
NW = NC * NS
N_PAD = 10240
STRIPE = N_PAD // NS
E = 320000 + N
K = 128
G = 81
G_AL = G + 1
E_PAD = NW * K * G
DEGC = 16
RB = 2000
NB = N // RB

_mesh = plsc.VectorSubcoreMesh(core_axis_name="c", subcore_axis_name="s")


@functools.partial(
    pl.kernel,
    out_type=jax.ShapeDtypeStruct((NC, N_PAD, DEGC), jnp.float32),
    mesh=_mesh,
    scratch_types=[
        pltpu.VMEM((K,), jnp.int32),
        pltpu.VMEM((K, DEGC), jnp.float32),
        pltpu.VMEM_SHARED((N_PAD, DEGC), jnp.float32),
    ],
)
def _sc_deg(dst_hbm, ones_hbm, zdeg_hbm, out_hbm, dst_v, ones_v, acc_sh):
    c = lax.axis_index("c")
    s = lax.axis_index("s")
    wid = c * NS + s
    pltpu.sync_copy(ones_hbm, ones_v)
    pltpu.sync_copy(zdeg_hbm, acc_sh.at[pl.ds(s * STRIPE, STRIPE)])
    plsc.subcore_barrier()
    base0 = wid * (G * K)

    def body(g, carry):
        pltpu.sync_copy(dst_hbm.at[pl.ds(base0 + g * K, K)], dst_v)
        pltpu.sync_copy(ones_v, acc_sh.at[dst_v], add=True)
        return carry

    lax.fori_loop(0, G, body, 0)
    plsc.subcore_barrier()
    pltpu.sync_copy(acc_sh.at[pl.ds(s * STRIPE, STRIPE)],
                    out_hbm.at[c, pl.ds(s * STRIPE, STRIPE)])


@functools.partial(
    pl.kernel,
    out_type=jax.ShapeDtypeStruct((NC, N_PAD, D), jnp.float32),
    mesh=_mesh,
    scratch_types=[
        pltpu.VMEM((K,), jnp.int32),
        pltpu.VMEM((K,), jnp.int32),
        pltpu.VMEM((K,), jnp.int32),
        pltpu.VMEM((K,), jnp.int32),
        pltpu.VMEM((K, D), jnp.float32),
        pltpu.VMEM((K, D), jnp.float32),
        pltpu.VMEM_SHARED((N_PAD, D), jnp.float32),
        pltpu.SemaphoreType.DMA,
        pltpu.SemaphoreType.DMA,
        pltpu.SemaphoreType.DMA,
    ],
)
def _sc_agg(h_hbm, src_hbm, dst_hbm, zrow_hbm, out_hbm,
            sidx0, sidx1, didx0, didx1, rows0, rows1, acc_sh,
            gsem0, gsem1, ssem):
    c = lax.axis_index("c")
    s = lax.axis_index("s")
    wid = c * NS + s
    pltpu.sync_copy(zrow_hbm, acc_sh.at[pl.ds(s * STRIPE, STRIPE)])
    plsc.subcore_barrier()
    base0 = wid * (G * K)

    def body(t, carry):
        ba = base0 + 2 * t * K
        bb = ba + K
        pltpu.sync_copy(src_hbm.at[pl.ds(ba, K)], sidx0)
        gather_a = pltpu.async_copy(h_hbm.at[sidx0], rows0, gsem0)
        pltpu.sync_copy(dst_hbm.at[pl.ds(ba, K)], didx0)
        pltpu.sync_copy(src_hbm.at[pl.ds(bb, K)], sidx1)
        gather_a.wait()
        scat_a = pltpu.async_copy(rows0, acc_sh.at[didx0], ssem, add=True)
        gather_b = pltpu.async_copy(h_hbm.at[sidx1], rows1, gsem1)
        pltpu.sync_copy(dst_hbm.at[pl.ds(bb, K)], didx1)
        scat_a.wait()
        gather_b.wait()
        pltpu.sync_copy(rows1, acc_sh.at[didx1], add=True)
        return carry

    lax.fori_loop(0, G // 2, body, 0)
    base = base0 + (G - 1) * K
    pltpu.sync_copy(src_hbm.at[pl.ds(base, K)], sidx0)
    gather = pltpu.async_copy(h_hbm.at[sidx0], rows0, gsem0)
    pltpu.sync_copy(dst_hbm.at[pl.ds(base, K)], didx0)
    gather.wait()
    pltpu.sync_copy(rows0, acc_sh.at[didx0], add=True)
    plsc.subcore_barrier()
    pltpu.sync_copy(acc_sh.at[pl.ds(s * STRIPE, STRIPE)],
                    out_hbm.at[c, pl.ds(s * STRIPE, STRIPE)])


def _dinv(degp_ref):
    return lax.rsqrt(degp_ref[0, :, :1] + degp_ref[1, :, :1])


def _tc_in_body(x_ref, w_ref, degp_ref, o_ref):
    o_ref[...] = _dinv(degp_ref) * jnp.dot(
        x_ref[...], w_ref[...], preferred_element_type=jnp.float32)


_tc_in = pl.pallas_call(
    _tc_in_body,
    grid=(NB,),
    in_specs=[
        pl.BlockSpec((RB, D), lambda i: (i, 0)),
        pl.BlockSpec((D, D), lambda i: (0, 0)),
        pl.BlockSpec((NC, RB, DEGC), lambda i: (0, i, 0)),
    ],
    out_specs=pl.BlockSpec((RB, D), lambda i: (i, 0)),
    out_shape=jax.ShapeDtypeStruct((N, D), jnp.float32),
)


def _tc_mid_body(p_ref, degp_ref, b1_ref, w2_ref, o_ref):
    dinv = _dinv(degp_ref)
    h2 = jnp.maximum(dinv * (p_ref[0] + p_ref[1]) + b1_ref[...], 0.0)
    o_ref[...] = dinv * jnp.dot(h2, w2_ref[...],
                                preferred_element_type=jnp.float32)


_tc_mid = pl.pallas_call(
    _tc_mid_body,
    grid=(NB,),
    in_specs=[
        pl.BlockSpec((NC, RB, D), lambda i: (0, i, 0)),
        pl.BlockSpec((NC, RB, DEGC), lambda i: (0, i, 0)),
        pl.BlockSpec((1, D), lambda i: (0, 0)),
        pl.BlockSpec((D, D), lambda i: (0, 0)),
    ],
    out_specs=pl.BlockSpec((RB, D), lambda i: (i, 0)),
    out_shape=jax.ShapeDtypeStruct((N, D), jnp.float32),
)


def _tc_out_body(q_ref, degp_ref, b2_ref, o_ref):
    o_ref[...] = _dinv(degp_ref) * (q_ref[0] + q_ref[1]) + b2_ref[...]


_tc_out = pl.pallas_call(
    _tc_out_body,
    grid=(NB,),
    in_specs=[
        pl.BlockSpec((NC, RB, D), lambda i: (0, i, 0)),
        pl.BlockSpec((NC, RB, DEGC), lambda i: (0, i, 0)),
        pl.BlockSpec((1, D), lambda i: (0, 0)),
    ],
    out_specs=pl.BlockSpec((RB, D), lambda i: (i, 0)),
    out_shape=jax.ShapeDtypeStruct((N, D), jnp.float32),
)


def kernel(x, edge_index, W1, b1, W2, b2):
    ei = edge_index.astype(jnp.int32)
    loop = jnp.arange(N, dtype=jnp.int32)
    pad = E_PAD - E
    src = jnp.concatenate([ei[0], loop, jnp.zeros((pad,), jnp.int32)])
    dst = jnp.concatenate([ei[1], loop, jnp.full((pad,), N, jnp.int32)])
    ones_blk = jnp.ones((K, DEGC), jnp.float32)
    zdeg = jnp.zeros((STRIPE, DEGC), jnp.float32)
    zrow = jnp.zeros((STRIPE, D), jnp.float32)

    degp = _sc_deg(dst, ones_blk, zdeg)
    h1 = _tc_in(x, W1, degp)
    p = _sc_agg(h1, src, dst, zrow)
    h3 = _tc_mid(p, degp, b1.reshape(1, D), W2)
    q = _sc_agg(h3, src, dst, zrow)
    return _tc_out(q, degp, b2.reshape(1, D))


# speedup vs baseline: 1.9619x; 1.0852x over previous
import functools

import jax
import jax.numpy as jnp
from jax import lax
from jax.experimental import pallas as pl
from jax.experimental.pallas import tpu as pltpu
from jax.experimental.pallas import tpu_sc as plsc

N = 10000
D = 128
NC = 2
NS = 16
NW = NC * NS
N_PAD = 10240
STRIPE = N_PAD // NS
E = 320000 + N
K = 128
G = 81
G_AL = G + 1
E_PAD = NW * K * G
DEGC = 16
RB = 2000
NB = N // RB

_mesh = plsc.VectorSubcoreMesh(core_axis_name="c", subcore_axis_name="s")


@functools.partial(
    pl.kernel,
    out_type=jax.ShapeDtypeStruct((NC, N_PAD, DEGC), jnp.float32),
    mesh=_mesh,
    scratch_types=[
        pltpu.VMEM((K,), jnp.int32),
        pltpu.VMEM((K, DEGC), jnp.float32),
        pltpu.VMEM_SHARED((N_PAD, DEGC), jnp.float32),
    ],
)
def _sc_deg(dst_hbm, ones_hbm, zdeg_hbm, out_hbm, dst_v, ones_v, acc_sh):
    c = lax.axis_index("c")
    s = lax.axis_index("s")
    wid = c * NS + s
    pltpu.sync_copy(ones_hbm, ones_v)
    pltpu.sync_copy(zdeg_hbm, acc_sh.at[pl.ds(s * STRIPE, STRIPE)])
    plsc.subcore_barrier()
    base0 = wid * (G * K)

    def body(g, carry):
        pltpu.sync_copy(dst_hbm.at[pl.ds(base0 + g * K, K)], dst_v)
        pltpu.sync_copy(ones_v, acc_sh.at[dst_v], add=True)
        return carry

    lax.fori_loop(0, G, body, 0)
    plsc.subcore_barrier()
    pltpu.sync_copy(acc_sh.at[pl.ds(s * STRIPE, STRIPE)],
                    out_hbm.at[c, pl.ds(s * STRIPE, STRIPE)])


@functools.partial(
    pl.kernel,
    out_type=jax.ShapeDtypeStruct((NC, N_PAD, D), jnp.float32),
    mesh=_mesh,
    scratch_types=[
        pltpu.VMEM((K,), jnp.int32),
        pltpu.VMEM((K,), jnp.int32),
        pltpu.VMEM((K,), jnp.int32),
        pltpu.VMEM((K,), jnp.int32),
        pltpu.VMEM((K, D), jnp.float32),
        pltpu.VMEM((K, D), jnp.float32),
        pltpu.VMEM_SHARED((N_PAD, D), jnp.float32),
        pltpu.SemaphoreType.DMA,
        pltpu.SemaphoreType.DMA,
        pltpu.SemaphoreType.DMA,
    ],
)
def _sc_agg(h_hbm, src_hbm, dst_hbm, zrow_hbm, out_hbm,
            sidx0, sidx1, didx0, didx1, rows0, rows1, acc_sh,
            gsem0, gsem1, ssem):
    c = lax.axis_index("c")
    s = lax.axis_index("s")
    wid = c * NS + s
    pltpu.sync_copy(zrow_hbm, acc_sh.at[pl.ds(s * STRIPE, STRIPE)])
    plsc.subcore_barrier()
    base0 = wid * (G * K)

    sidx = (sidx0, sidx1)
    didx = (didx0, didx1)
    rows = (rows0, rows1)
    U = 8

    def body(t, carry):
        c0 = base0 + t * U * K
        pltpu.sync_copy(src_hbm.at[pl.ds(c0, K)], sidx0)
        g = pltpu.async_copy(h_hbm.at[sidx0], rows0, gsem0)
        pltpu.sync_copy(dst_hbm.at[pl.ds(c0, K)], didx0)
        pltpu.sync_copy(src_hbm.at[pl.ds(c0 + K, K)], sidx1)
        g.wait()
        for u in range(1, U):
            bu = u % 2
            sc = pltpu.async_copy(rows[1 - bu], acc_sh.at[didx[1 - bu]],
                                  ssem, add=True)
            g = pltpu.async_copy(h_hbm.at[sidx[bu]], rows[bu], gsem0)
            pltpu.sync_copy(dst_hbm.at[pl.ds(c0 + u * K, K)], didx[bu])
            if u + 1 < U:
                pltpu.sync_copy(src_hbm.at[pl.ds(c0 + (u + 1) * K, K)],
                                sidx[1 - bu])
            sc.wait()
            g.wait()
        pltpu.sync_copy(rows[(U - 1) % 2], acc_sh.at[didx[(U - 1) % 2]],
                        add=True)
        return carry

    lax.fori_loop(0, G // U, body, 0)
    for gch in range((G // U) * U, G):
        base = base0 + gch * K
        pltpu.sync_copy(src_hbm.at[pl.ds(base, K)], sidx0)
        g = pltpu.async_copy(h_hbm.at[sidx0], rows0, gsem0)
        pltpu.sync_copy(dst_hbm.at[pl.ds(base, K)], didx0)
        g.wait()
        pltpu.sync_copy(rows0, acc_sh.at[didx0], add=True)
    plsc.subcore_barrier()
    pltpu.sync_copy(acc_sh.at[pl.ds(s * STRIPE, STRIPE)],
                    out_hbm.at[c, pl.ds(s * STRIPE, STRIPE)])


def _dinv(degp_ref):
    return lax.rsqrt(degp_ref[0, :, :1] + degp_ref[1, :, :1])


def _tc_in_body(x_ref, w_ref, degp_ref, o_ref):
    o_ref[...] = _dinv(degp_ref) * jnp.dot(
        x_ref[...], w_ref[...], preferred_element_type=jnp.float32)


_tc_in = pl.pallas_call(
    _tc_in_body,
    grid=(NB,),
    in_specs=[
        pl.BlockSpec((RB, D), lambda i: (i, 0)),
        pl.BlockSpec((D, D), lambda i: (0, 0)),
        pl.BlockSpec((NC, RB, DEGC), lambda i: (0, i, 0)),
    ],
    out_specs=pl.BlockSpec((RB, D), lambda i: (i, 0)),
    out_shape=jax.ShapeDtypeStruct((N, D), jnp.float32),
)


def _tc_mid_body(p_ref, degp_ref, b1_ref, w2_ref, o_ref):
    dinv = _dinv(degp_ref)
    h2 = jnp.maximum(dinv * (p_ref[0] + p_ref[1]) + b1_ref[...], 0.0)
    o_ref[...] = dinv * jnp.dot(h2, w2_ref[...],
                                preferred_element_type=jnp.float32)


_tc_mid = pl.pallas_call(
    _tc_mid_body,
    grid=(NB,),
    in_specs=[
        pl.BlockSpec((NC, RB, D), lambda i: (0, i, 0)),
        pl.BlockSpec((NC, RB, DEGC), lambda i: (0, i, 0)),
        pl.BlockSpec((1, D), lambda i: (0, 0)),
        pl.BlockSpec((D, D), lambda i: (0, 0)),
    ],
    out_specs=pl.BlockSpec((RB, D), lambda i: (i, 0)),
    out_shape=jax.ShapeDtypeStruct((N, D), jnp.float32),
)


def _tc_out_body(q_ref, degp_ref, b2_ref, o_ref):
    o_ref[...] = _dinv(degp_ref) * (q_ref[0] + q_ref[1]) + b2_ref[...]


_tc_out = pl.pallas_call(
    _tc_out_body,
    grid=(NB,),
    in_specs=[
        pl.BlockSpec((NC, RB, D), lambda i: (0, i, 0)),
        pl.BlockSpec((NC, RB, DEGC), lambda i: (0, i, 0)),
        pl.BlockSpec((1, D), lambda i: (0, 0)),
    ],
    out_specs=pl.BlockSpec((RB, D), lambda i: (i, 0)),
    out_shape=jax.ShapeDtypeStruct((N, D), jnp.float32),
)


def kernel(x, edge_index, W1, b1, W2, b2):
    ei = edge_index.astype(jnp.int32)
    loop = jnp.arange(N, dtype=jnp.int32)
    pad = E_PAD - E
    src = jnp.concatenate([ei[0], loop, jnp.zeros((pad,), jnp.int32)])
    dst = jnp.concatenate([ei[1], loop, jnp.full((pad,), N, jnp.int32)])
    ones_blk = jnp.ones((K, DEGC), jnp.float32)
    zdeg = jnp.zeros((STRIPE, DEGC), jnp.float32)
    zrow = jnp.zeros((STRIPE, D), jnp.float32)

    degp = _sc_deg(dst, ones_blk, zdeg)
    h1 = _tc_in(x, W1, degp)
    p = _sc_agg(h1, src, dst, zrow)
    h3 = _tc_mid(p, degp, b1.reshape(1, D), W2)
    q = _sc_agg(h3, src, dst, zrow)
    return _tc_out(q, degp, b2.reshape(1, D))


# speedup vs baseline: 1.9762x; 1.0073x over previous
import functools

import jax
import jax.numpy as jnp
from jax import lax
from jax.experimental import pallas as pl
from jax.experimental.pallas import tpu as pltpu
from jax.experimental.pallas import tpu_sc as plsc

N = 10000
D = 128
NC = 2
NS = 16
NW = NC * NS
N_PAD = 10240
STRIPE = N_PAD // NS
E = 320000 + N
K = 128
G = 81
G_AL = G + 1
E_PAD = NW * K * G
DEGC = 16
RB = 2000
NB = N // RB

_mesh = plsc.VectorSubcoreMesh(core_axis_name="c", subcore_axis_name="s")


@functools.partial(
    pl.kernel,
    out_type=jax.ShapeDtypeStruct((NC, N_PAD, DEGC), jnp.float32),
    mesh=_mesh,
    scratch_types=[
        pltpu.VMEM((K,), jnp.int32),
        pltpu.VMEM((K,), jnp.int32),
        pltpu.VMEM((K, DEGC), jnp.float32),
        pltpu.VMEM_SHARED((N_PAD, DEGC), jnp.float32),
        pltpu.SemaphoreType.DMA,
    ],
)
def _sc_deg(dst_hbm, ones_hbm, zdeg_hbm, out_hbm, didx0, didx1, ones_v,
            acc_sh, ssem):
    c = lax.axis_index("c")
    s = lax.axis_index("s")
    wid = c * NS + s
    pltpu.sync_copy(ones_hbm, ones_v)
    pltpu.sync_copy(zdeg_hbm, acc_sh.at[pl.ds(s * STRIPE, STRIPE)])
    plsc.subcore_barrier()
    base0 = wid * (G * K)
    didx = (didx0, didx1)
    UD = 16

    def body(t, carry):
        c0 = base0 + t * UD * K
        pltpu.sync_copy(dst_hbm.at[pl.ds(c0, K)], didx0)
        for u in range(1, UD):
            bu = u % 2
            sc = pltpu.async_copy(ones_v, acc_sh.at[didx[1 - bu]],
                                  ssem, add=True)
            pltpu.sync_copy(dst_hbm.at[pl.ds(c0 + u * K, K)], didx[bu])
            sc.wait()
        pltpu.sync_copy(ones_v, acc_sh.at[didx[(UD - 1) % 2]], add=True)
        return carry

    lax.fori_loop(0, G // UD, body, 0)
    for gch in range((G // UD) * UD, G):
        pltpu.sync_copy(dst_hbm.at[pl.ds(base0 + gch * K, K)], didx0)
        pltpu.sync_copy(ones_v, acc_sh.at[didx0], add=True)
    plsc.subcore_barrier()
    pltpu.sync_copy(acc_sh.at[pl.ds(s * STRIPE, STRIPE)],
                    out_hbm.at[c, pl.ds(s * STRIPE, STRIPE)])


@functools.partial(
    pl.kernel,
    out_type=jax.ShapeDtypeStruct((NC, N_PAD, D), jnp.float32),
    mesh=_mesh,
    scratch_types=[
        pltpu.VMEM((K,), jnp.int32),
        pltpu.VMEM((K,), jnp.int32),
        pltpu.VMEM((K,), jnp.int32),
        pltpu.VMEM((K,), jnp.int32),
        pltpu.VMEM((K, D), jnp.float32),
        pltpu.VMEM((K, D), jnp.float32),
        pltpu.VMEM_SHARED((N_PAD, D), jnp.float32),
        pltpu.SemaphoreType.DMA,
        pltpu.SemaphoreType.DMA,
        pltpu.SemaphoreType.DMA,
    ],
)
def _sc_agg(h_hbm, src_hbm, dst_hbm, zrow_hbm, out_hbm,
            sidx0, sidx1, didx0, didx1, rows0, rows1, acc_sh,
            gsem0, gsem1, ssem):
    c = lax.axis_index("c")
    s = lax.axis_index("s")
    wid = c * NS + s
    pltpu.sync_copy(zrow_hbm, acc_sh.at[pl.ds(s * STRIPE, STRIPE)])
    plsc.subcore_barrier()
    base0 = wid * (G * K)

    sidx = (sidx0, sidx1)
    didx = (didx0, didx1)
    rows = (rows0, rows1)
    U = 16

    def body(t, carry):
        c0 = base0 + t * U * K
        pltpu.sync_copy(src_hbm.at[pl.ds(c0, K)], sidx0)
        g = pltpu.async_copy(h_hbm.at[sidx0], rows0, gsem0)
        pltpu.sync_copy(dst_hbm.at[pl.ds(c0, K)], didx0)
        pltpu.sync_copy(src_hbm.at[pl.ds(c0 + K, K)], sidx1)
        g.wait()
        for u in range(1, U):
            bu = u % 2
            sc = pltpu.async_copy(rows[1 - bu], acc_sh.at[didx[1 - bu]],
                                  ssem, add=True)
            g = pltpu.async_copy(h_hbm.at[sidx[bu]], rows[bu], gsem0)
            pltpu.sync_copy(dst_hbm.at[pl.ds(c0 + u * K, K)], didx[bu])
            if u + 1 < U:
                pltpu.sync_copy(src_hbm.at[pl.ds(c0 + (u + 1) * K, K)],
                                sidx[1 - bu])
            sc.wait()
            g.wait()
        pltpu.sync_copy(rows[(U - 1) % 2], acc_sh.at[didx[(U - 1) % 2]],
                        add=True)
        return carry

    lax.fori_loop(0, G // U, body, 0)
    for gch in range((G // U) * U, G):
        base = base0 + gch * K
        pltpu.sync_copy(src_hbm.at[pl.ds(base, K)], sidx0)
        g = pltpu.async_copy(h_hbm.at[sidx0], rows0, gsem0)
        pltpu.sync_copy(dst_hbm.at[pl.ds(base, K)], didx0)
        g.wait()
        pltpu.sync_copy(rows0, acc_sh.at[didx0], add=True)
    plsc.subcore_barrier()
    pltpu.sync_copy(acc_sh.at[pl.ds(s * STRIPE, STRIPE)],
                    out_hbm.at[c, pl.ds(s * STRIPE, STRIPE)])


def _dinv(degp_ref):
    return lax.rsqrt(degp_ref[0, :, :1] + degp_ref[1, :, :1])


def _tc_in_body(x_ref, w_ref, degp_ref, o_ref):
    o_ref[...] = _dinv(degp_ref) * jnp.dot(
        x_ref[...], w_ref[...], preferred_element_type=jnp.float32)


_tc_in = pl.pallas_call(
    _tc_in_body,
    grid=(NB,),
    in_specs=[
        pl.BlockSpec((RB, D), lambda i: (i, 0)),
        pl.BlockSpec((D, D), lambda i: (0, 0)),
        pl.BlockSpec((NC, RB, DEGC), lambda i: (0, i, 0)),
    ],
    out_specs=pl.BlockSpec((RB, D), lambda i: (i, 0)),
    out_shape=jax.ShapeDtypeStruct((N, D), jnp.float32),
)


def _tc_mid_body(p_ref, degp_ref, b1_ref, w2_ref, o_ref):
    dinv = _dinv(degp_ref)
    h2 = jnp.maximum(dinv * (p_ref[0] + p_ref[1]) + b1_ref[...], 0.0)
    o_ref[...] = dinv * jnp.dot(h2, w2_ref[...],
                                preferred_element_type=jnp.float32)


_tc_mid = pl.pallas_call(
    _tc_mid_body,
    grid=(NB,),
    in_specs=[
        pl.BlockSpec((NC, RB, D), lambda i: (0, i, 0)),
        pl.BlockSpec((NC, RB, DEGC), lambda i: (0, i, 0)),
        pl.BlockSpec((1, D), lambda i: (0, 0)),
        pl.BlockSpec((D, D), lambda i: (0, 0)),
    ],
    out_specs=pl.BlockSpec((RB, D), lambda i: (i, 0)),
    out_shape=jax.ShapeDtypeStruct((N, D), jnp.float32),
)


def _tc_out_body(q_ref, degp_ref, b2_ref, o_ref):
    o_ref[...] = _dinv(degp_ref) * (q_ref[0] + q_ref[1]) + b2_ref[...]


_tc_out = pl.pallas_call(
    _tc_out_body,
    grid=(NB,),
    in_specs=[
        pl.BlockSpec((NC, RB, D), lambda i: (0, i, 0)),
        pl.BlockSpec((NC, RB, DEGC), lambda i: (0, i, 0)),
        pl.BlockSpec((1, D), lambda i: (0, 0)),
    ],
    out_specs=pl.BlockSpec((RB, D), lambda i: (i, 0)),
    out_shape=jax.ShapeDtypeStruct((N, D), jnp.float32),
)


def kernel(x, edge_index, W1, b1, W2, b2):
    ei = edge_index.astype(jnp.int32)
    loop = jnp.arange(N, dtype=jnp.int32)
    pad = E_PAD - E
    src = jnp.concatenate([ei[0], loop, jnp.zeros((pad,), jnp.int32)])
    dst = jnp.concatenate([ei[1], loop, jnp.full((pad,), N, jnp.int32)])
    ones_blk = jnp.ones((K, DEGC), jnp.float32)
    zdeg = jnp.zeros((STRIPE, DEGC), jnp.float32)
    zrow = jnp.zeros((STRIPE, D), jnp.float32)

    degp = _sc_deg(dst, ones_blk, zdeg)
    h1 = _tc_in(x, W1, degp)
    p = _sc_agg(h1, src, dst, zrow)
    h3 = _tc_mid(p, degp, b1.reshape(1, D), W2)
    q = _sc_agg(h3, src, dst, zrow)
    return _tc_out(q, degp, b2.reshape(1, D))


# speedup vs baseline: 2.0071x; 1.0156x over previous
import functools

import jax
import jax.numpy as jnp
from jax import lax
from jax.experimental import pallas as pl
from jax.experimental.pallas import tpu as pltpu
from jax.experimental.pallas import tpu_sc as plsc

N = 10000
D = 128
NC = 2
NS = 16
NW = NC * NS
N_PAD = 10240
STRIPE = N_PAD // NS
E = 320000 + N
K = 128
G = 81
G_AL = G + 1
E_PAD = NW * K * G
DEGC = 1
RB = 2000
NB = N // RB

_mesh = plsc.VectorSubcoreMesh(core_axis_name="c", subcore_axis_name="s")


@functools.partial(
    pl.kernel,
    out_type=jax.ShapeDtypeStruct((NC, N_PAD, DEGC), jnp.float32),
    mesh=_mesh,
    scratch_types=[
        pltpu.VMEM((K,), jnp.int32),
        pltpu.VMEM((K,), jnp.int32),
        pltpu.VMEM((K, DEGC), jnp.float32),
        pltpu.VMEM_SHARED((N_PAD, DEGC), jnp.float32),
        pltpu.SemaphoreType.DMA,
    ],
)
def _sc_deg(dst_hbm, ones_hbm, zdeg_hbm, out_hbm, didx0, didx1, ones_v,
            acc_sh, ssem):
    c = lax.axis_index("c")
    s = lax.axis_index("s")
    wid = c * NS + s
    pltpu.sync_copy(ones_hbm, ones_v)
    pltpu.sync_copy(zdeg_hbm, acc_sh.at[pl.ds(s * STRIPE, STRIPE)])
    plsc.subcore_barrier()
    base0 = wid * (G * K)
    didx = (didx0, didx1)
    UD = 16

    def body(t, carry):
        c0 = base0 + t * UD * K
        pltpu.sync_copy(dst_hbm.at[pl.ds(c0, K)], didx0)
        for u in range(1, UD):
            bu = u % 2
            sc = pltpu.async_copy(ones_v, acc_sh.at[didx[1 - bu]],
                                  ssem, add=True)
            pltpu.sync_copy(dst_hbm.at[pl.ds(c0 + u * K, K)], didx[bu])
            sc.wait()
        pltpu.sync_copy(ones_v, acc_sh.at[didx[(UD - 1) % 2]], add=True)
        return carry

    lax.fori_loop(0, G // UD, body, 0)
    for gch in range((G // UD) * UD, G):
        pltpu.sync_copy(dst_hbm.at[pl.ds(base0 + gch * K, K)], didx0)
        pltpu.sync_copy(ones_v, acc_sh.at[didx0], add=True)
    plsc.subcore_barrier()
    pltpu.sync_copy(acc_sh.at[pl.ds(s * STRIPE, STRIPE)],
                    out_hbm.at[c, pl.ds(s * STRIPE, STRIPE)])


@functools.partial(
    pl.kernel,
    out_type=jax.ShapeDtypeStruct((NC, N_PAD, D), jnp.float32),
    mesh=_mesh,
    scratch_types=[
        pltpu.VMEM((K,), jnp.int32),
        pltpu.VMEM((K,), jnp.int32),
        pltpu.VMEM((K,), jnp.int32),
        pltpu.VMEM((K,), jnp.int32),
        pltpu.VMEM((K, D), jnp.float32),
        pltpu.VMEM((K, D), jnp.float32),
        pltpu.VMEM_SHARED((N_PAD, D), jnp.float32),
        pltpu.SemaphoreType.DMA,
        pltpu.SemaphoreType.DMA,
        pltpu.SemaphoreType.DMA,
    ],
)
def _sc_agg(h_hbm, src_hbm, dst_hbm, zrow_hbm, out_hbm,
            sidx0, sidx1, didx0, didx1, rows0, rows1, acc_sh,
            gsem0, gsem1, ssem):
    c = lax.axis_index("c")
    s = lax.axis_index("s")
    wid = c * NS + s
    pltpu.sync_copy(zrow_hbm, acc_sh.at[pl.ds(s * STRIPE, STRIPE)])
    plsc.subcore_barrier()
    base0 = wid * (G * K)

    sidx = (sidx0, sidx1)
    didx = (didx0, didx1)
    rows = (rows0, rows1)
    U = 16

    def body(t, carry):
        c0 = base0 + t * U * K
        pltpu.sync_copy(src_hbm.at[pl.ds(c0, K)], sidx0)
        g = pltpu.async_copy(h_hbm.at[sidx0], rows0, gsem0)
        pltpu.sync_copy(dst_hbm.at[pl.ds(c0, K)], didx0)
        pltpu.sync_copy(src_hbm.at[pl.ds(c0 + K, K)], sidx1)
        g.wait()
        for u in range(1, U):
            bu = u % 2
            sc = pltpu.async_copy(rows[1 - bu], acc_sh.at[didx[1 - bu]],
                                  ssem, add=True)
            g = pltpu.async_copy(h_hbm.at[sidx[bu]], rows[bu], gsem0)
            pltpu.sync_copy(dst_hbm.at[pl.ds(c0 + u * K, K)], didx[bu])
            if u + 1 < U:
                pltpu.sync_copy(src_hbm.at[pl.ds(c0 + (u + 1) * K, K)],
                                sidx[1 - bu])
            sc.wait()
            g.wait()
        pltpu.sync_copy(rows[(U - 1) % 2], acc_sh.at[didx[(U - 1) % 2]],
                        add=True)
        return carry

    lax.fori_loop(0, G // U, body, 0)
    for gch in range((G // U) * U, G):
        base = base0 + gch * K
        pltpu.sync_copy(src_hbm.at[pl.ds(base, K)], sidx0)
        g = pltpu.async_copy(h_hbm.at[sidx0], rows0, gsem0)
        pltpu.sync_copy(dst_hbm.at[pl.ds(base, K)], didx0)
        g.wait()
        pltpu.sync_copy(rows0, acc_sh.at[didx0], add=True)
    plsc.subcore_barrier()
    pltpu.sync_copy(acc_sh.at[pl.ds(s * STRIPE, STRIPE)],
                    out_hbm.at[c, pl.ds(s * STRIPE, STRIPE)])


def _dinv(degp_ref):
    return lax.rsqrt(degp_ref[0, :, :1] + degp_ref[1, :, :1])


def _tc_in_body(x_ref, w_ref, degp_ref, o_ref):
    o_ref[...] = _dinv(degp_ref) * jnp.dot(
        x_ref[...], w_ref[...], preferred_element_type=jnp.float32)


_tc_in = pl.pallas_call(
    _tc_in_body,
    grid=(NB,),
    in_specs=[
        pl.BlockSpec((RB, D), lambda i: (i, 0)),
        pl.BlockSpec((D, D), lambda i: (0, 0)),
        pl.BlockSpec((NC, RB, DEGC), lambda i: (0, i, 0)),
    ],
    out_specs=pl.BlockSpec((RB, D), lambda i: (i, 0)),
    out_shape=jax.ShapeDtypeStruct((N, D), jnp.float32),
)


def _tc_mid_body(p_ref, degp_ref, b1_ref, w2_ref, o_ref):
    dinv = _dinv(degp_ref)
    h2 = jnp.maximum(dinv * (p_ref[0] + p_ref[1]) + b1_ref[...], 0.0)
    o_ref[...] = dinv * jnp.dot(h2, w2_ref[...],
                                preferred_element_type=jnp.float32)


_tc_mid = pl.pallas_call(
    _tc_mid_body,
    grid=(NB,),
    in_specs=[
        pl.BlockSpec((NC, RB, D), lambda i: (0, i, 0)),
        pl.BlockSpec((NC, RB, DEGC), lambda i: (0, i, 0)),
        pl.BlockSpec((1, D), lambda i: (0, 0)),
        pl.BlockSpec((D, D), lambda i: (0, 0)),
    ],
    out_specs=pl.BlockSpec((RB, D), lambda i: (i, 0)),
    out_shape=jax.ShapeDtypeStruct((N, D), jnp.float32),
)


def _tc_out_body(q_ref, degp_ref, b2_ref, o_ref):
    o_ref[...] = _dinv(degp_ref) * (q_ref[0] + q_ref[1]) + b2_ref[...]


_tc_out = pl.pallas_call(
    _tc_out_body,
    grid=(NB,),
    in_specs=[
        pl.BlockSpec((NC, RB, D), lambda i: (0, i, 0)),
        pl.BlockSpec((NC, RB, DEGC), lambda i: (0, i, 0)),
        pl.BlockSpec((1, D), lambda i: (0, 0)),
    ],
    out_specs=pl.BlockSpec((RB, D), lambda i: (i, 0)),
    out_shape=jax.ShapeDtypeStruct((N, D), jnp.float32),
)


def kernel(x, edge_index, W1, b1, W2, b2):
    ei = edge_index.astype(jnp.int32)
    loop = jnp.arange(N, dtype=jnp.int32)
    pad = E_PAD - E
    src = jnp.concatenate([ei[0], loop, jnp.zeros((pad,), jnp.int32)])
    dst = jnp.concatenate([ei[1], loop, jnp.full((pad,), N, jnp.int32)])
    ones_blk = jnp.ones((K, DEGC), jnp.float32)
    zdeg = jnp.zeros((STRIPE, DEGC), jnp.float32)
    zrow = jnp.zeros((STRIPE, D), jnp.float32)

    degp = _sc_deg(dst, ones_blk, zdeg)
    h1 = _tc_in(x, W1, degp)
    p = _sc_agg(h1, src, dst, zrow)
    h3 = _tc_mid(p, degp, b1.reshape(1, D), W2)
    q = _sc_agg(h3, src, dst, zrow)
    return _tc_out(q, degp, b2.reshape(1, D))


# speedup vs baseline: 2.1335x; 1.0630x over previous
import functools

import jax
import jax.numpy as jnp
from jax import lax
from jax.experimental import pallas as pl
from jax.experimental.pallas import tpu as pltpu
from jax.experimental.pallas import tpu_sc as plsc

N = 10000
D = 128
NC = 2
NS = 16
NW = NC * NS
N_PAD = 10112
STRIPE = N_PAD // NS
E = 320000 + N
K = 128
G = 81
G_AL = G + 1
E_PAD = NW * K * G
DEGC = 1
RB = 2000
NB = N // RB

_mesh = plsc.VectorSubcoreMesh(core_axis_name="c", subcore_axis_name="s")


@functools.partial(
    pl.kernel,
    out_type=jax.ShapeDtypeStruct((NC, N_PAD, DEGC), jnp.float32),
    mesh=_mesh,
    scratch_types=[
        pltpu.VMEM((K,), jnp.int32),
        pltpu.VMEM((K,), jnp.int32),
        pltpu.VMEM((K, DEGC), jnp.float32),
        pltpu.VMEM_SHARED((N_PAD, DEGC), jnp.float32),
        pltpu.SemaphoreType.DMA,
    ],
)
def _sc_deg(dst_hbm, ones_hbm, zdeg_hbm, out_hbm, didx0, didx1, ones_v,
            acc_sh, ssem):
    c = lax.axis_index("c")
    s = lax.axis_index("s")
    wid = c * NS + s
    pltpu.sync_copy(ones_hbm, ones_v)
    pltpu.sync_copy(zdeg_hbm, acc_sh.at[pl.ds(s * STRIPE, STRIPE)])
    plsc.subcore_barrier()
    base0 = wid * (G * K)
    didx = (didx0, didx1)
    UD = 16

    def body(t, carry):
        c0 = base0 + t * UD * K
        pltpu.sync_copy(dst_hbm.at[pl.ds(c0, K)], didx0)
        for u in range(1, UD):
            bu = u % 2
            sc = pltpu.async_copy(ones_v, acc_sh.at[didx[1 - bu]],
                                  ssem, add=True)
            pltpu.sync_copy(dst_hbm.at[pl.ds(c0 + u * K, K)], didx[bu])
            sc.wait()
        pltpu.sync_copy(ones_v, acc_sh.at[didx[(UD - 1) % 2]], add=True)
        return carry

    lax.fori_loop(0, G // UD, body, 0)
    for gch in range((G // UD) * UD, G):
        pltpu.sync_copy(dst_hbm.at[pl.ds(base0 + gch * K, K)], didx0)
        pltpu.sync_copy(ones_v, acc_sh.at[didx0], add=True)
    plsc.subcore_barrier()
    pltpu.sync_copy(acc_sh.at[pl.ds(s * STRIPE, STRIPE)],
                    out_hbm.at[c, pl.ds(s * STRIPE, STRIPE)])


@functools.partial(
    pl.kernel,
    out_type=jax.ShapeDtypeStruct((NC, N_PAD, D), jnp.float32),
    mesh=_mesh,
    scratch_types=[
        pltpu.VMEM((K,), jnp.int32),
        pltpu.VMEM((K,), jnp.int32),
        pltpu.VMEM((K,), jnp.int32),
        pltpu.VMEM((K,), jnp.int32),
        pltpu.VMEM((K,), jnp.int32),
        pltpu.VMEM((K,), jnp.int32),
        pltpu.VMEM((K, D), jnp.float32),
        pltpu.VMEM((K, D), jnp.float32),
        pltpu.VMEM((K, D), jnp.float32),
        pltpu.VMEM_SHARED((N_PAD, D), jnp.float32),
        pltpu.SemaphoreType.DMA,
        pltpu.SemaphoreType.DMA,
        pltpu.SemaphoreType.DMA,
    ],
)
def _sc_agg(h_hbm, src_hbm, dst_hbm, zrow_hbm, out_hbm,
            sidx0, sidx1, sidx2, didx0, didx1, didx2,
            rows0, rows1, rows2, acc_sh, gsem0, gsem1, ssem):
    c = lax.axis_index("c")
    s = lax.axis_index("s")
    wid = c * NS + s
    pltpu.sync_copy(zrow_hbm, acc_sh.at[pl.ds(s * STRIPE, STRIPE)])
    plsc.subcore_barrier()
    base0 = wid * (G * K)

    sidx = (sidx0, sidx1, sidx2)
    didx = (didx0, didx1, didx2)
    rows = (rows0, rows1, rows2)
    gsem = (gsem0, gsem1)
    U = 16

    def body(t, carry):
        c0 = base0 + t * U * K

        def src_at(i):
            return src_hbm.at[pl.ds(c0 + i * K, K)]

        def dst_at(i):
            return dst_hbm.at[pl.ds(c0 + i * K, K)]

        pltpu.sync_copy(src_at(0), sidx[0])
        pltpu.sync_copy(dst_at(0), didx[0])
        g_cur = pltpu.async_copy(h_hbm.at[sidx[0]], rows[0], gsem[0])
        pltpu.sync_copy(src_at(1), sidx[1])
        pltpu.sync_copy(dst_at(1), didx[1])
        g_nxt = pltpu.async_copy(h_hbm.at[sidx[1]], rows[1], gsem[1])
        pltpu.sync_copy(src_at(2), sidx[2])
        g_cur.wait()
        for u in range(1, U):
            sc = pltpu.async_copy(rows[(u - 1) % 3],
                                  acc_sh.at[didx[(u - 1) % 3]],
                                  ssem, add=True)
            g_cur = g_nxt
            if u + 1 < U:
                g_nxt = pltpu.async_copy(h_hbm.at[sidx[(u + 1) % 3]],
                                         rows[(u + 1) % 3],
                                         gsem[(u + 1) % 2])
            if u + 2 < U:
                pltpu.sync_copy(src_at(u + 2), sidx[(u + 2) % 3])
            if u + 1 < U:
                pltpu.sync_copy(dst_at(u + 1), didx[(u + 1) % 3])
            sc.wait()
            g_cur.wait()
        pltpu.sync_copy(rows[(U - 1) % 3], acc_sh.at[didx[(U - 1) % 3]],
                        add=True)
        return carry

    lax.fori_loop(0, G // U, body, 0)
    for gch in range((G // U) * U, G):
        base = base0 + gch * K
        pltpu.sync_copy(src_hbm.at[pl.ds(base, K)], sidx0)
        g = pltpu.async_copy(h_hbm.at[sidx0], rows0, gsem0)
        pltpu.sync_copy(dst_hbm.at[pl.ds(base, K)], didx0)
        g.wait()
        pltpu.sync_copy(rows0, acc_sh.at[didx0], add=True)
    plsc.subcore_barrier()
    pltpu.sync_copy(acc_sh.at[pl.ds(s * STRIPE, STRIPE)],
                    out_hbm.at[c, pl.ds(s * STRIPE, STRIPE)])


def _dinv(degp_ref):
    return lax.rsqrt(degp_ref[0, :, :1] + degp_ref[1, :, :1])


def _tc_in_body(x_ref, w_ref, degp_ref, o_ref):
    o_ref[...] = _dinv(degp_ref) * jnp.dot(
        x_ref[...], w_ref[...], preferred_element_type=jnp.float32)


_tc_in = pl.pallas_call(
    _tc_in_body,
    grid=(NB,),
    in_specs=[
        pl.BlockSpec((RB, D), lambda i: (i, 0)),
        pl.BlockSpec((D, D), lambda i: (0, 0)),
        pl.BlockSpec((NC, RB, DEGC), lambda i: (0, i, 0)),
    ],
    out_specs=pl.BlockSpec((RB, D), lambda i: (i, 0)),
    out_shape=jax.ShapeDtypeStruct((N, D), jnp.float32),
)


def _tc_mid_body(p_ref, degp_ref, b1_ref, w2_ref, o_ref):
    dinv = _dinv(degp_ref)
    h2 = jnp.maximum(dinv * (p_ref[0] + p_ref[1]) + b1_ref[...], 0.0)
    o_ref[...] = dinv * jnp.dot(h2, w2_ref[...],
                                preferred_element_type=jnp.float32)


_tc_mid = pl.pallas_call(
    _tc_mid_body,
    grid=(NB,),
    in_specs=[
        pl.BlockSpec((NC, RB, D), lambda i: (0, i, 0)),
        pl.BlockSpec((NC, RB, DEGC), lambda i: (0, i, 0)),
        pl.BlockSpec((1, D), lambda i: (0, 0)),
        pl.BlockSpec((D, D), lambda i: (0, 0)),
    ],
    out_specs=pl.BlockSpec((RB, D), lambda i: (i, 0)),
    out_shape=jax.ShapeDtypeStruct((N, D), jnp.float32),
)


def _tc_out_body(q_ref, degp_ref, b2_ref, o_ref):
    o_ref[...] = _dinv(degp_ref) * (q_ref[0] + q_ref[1]) + b2_ref[...]


_tc_out = pl.pallas_call(
    _tc_out_body,
    grid=(NB,),
    in_specs=[
        pl.BlockSpec((NC, RB, D), lambda i: (0, i, 0)),
        pl.BlockSpec((NC, RB, DEGC), lambda i: (0, i, 0)),
        pl.BlockSpec((1, D), lambda i: (0, 0)),
    ],
    out_specs=pl.BlockSpec((RB, D), lambda i: (i, 0)),
    out_shape=jax.ShapeDtypeStruct((N, D), jnp.float32),
)


def kernel(x, edge_index, W1, b1, W2, b2):
    ei = edge_index.astype(jnp.int32)
    loop = jnp.arange(N, dtype=jnp.int32)
    pad = E_PAD - E
    src = jnp.concatenate([ei[0], loop, jnp.zeros((pad,), jnp.int32)])
    dst = jnp.concatenate([ei[1], loop, jnp.full((pad,), N, jnp.int32)])
    ones_blk = jnp.ones((K, DEGC), jnp.float32)
    zdeg = jnp.zeros((STRIPE, DEGC), jnp.float32)
    zrow = jnp.zeros((STRIPE, D), jnp.float32)

    degp = _sc_deg(dst, ones_blk, zdeg)
    h1 = _tc_in(x, W1, degp)
    p = _sc_agg(h1, src, dst, zrow)
    h3 = _tc_mid(p, degp, b1.reshape(1, D), W2)
    q = _sc_agg(h3, src, dst, zrow)
    return _tc_out(q, degp, b2.reshape(1, D))
